# baseline (speedup 1.0000x reference)
import functools
import math

import jax
import jax.numpy as jnp
from jax import lax
from jax.experimental import pallas as pl
from jax.experimental.pallas import tpu as pltpu
from jax.experimental.pallas import tpu_sc as plsc

D_MODEL = 128
SCALE = math.sqrt(D_MODEL)

_info = plsc.get_sparse_core_info()
NC, NS, L = _info.num_cores, _info.num_subcores, _info.num_lanes
NW = NC * NS

NBUF = 5
LA = 3


def _make_kernel(n_rows, seq):
    assert n_rows % NW == 0
    cols_per_w = n_rows // NW
    assert seq % NBUF == 0
    mesh = plsc.VectorSubcoreMesh(core_axis_name="c", subcore_axis_name="s")

    @functools.partial(
        pl.kernel,
        mesh=mesh,
        out_type=jax.ShapeDtypeStruct((seq, n_rows, D_MODEL), jnp.float32),
        compiler_params=pltpu.CompilerParams(use_tc_tiling_on_sc=True),
        scratch_types=(
            [pltpu.VMEM((seq, cols_per_w), jnp.int32)]
            + [pltpu.VMEM((cols_per_w, D_MODEL), jnp.float32)] * NBUF
            + [pltpu.SemaphoreType.DMA] * (2 * NBUF)
        ),
    )
    def k(xt_hbm, table_hbm, out_hbm, idx_v, *rest):
        bufs = rest[:NBUF]
        gsems = rest[NBUF:2 * NBUF]
        osems = rest[2 * NBUF:3 * NBUF]
        wid = lax.axis_index("s") * NC + lax.axis_index("c")
        col0 = wid * cols_per_w
        pltpu.sync_copy(xt_hbm.at[:, pl.ds(col0, cols_per_w)], idx_v)

        def issue_gather(s, b):
            pltpu.async_copy(table_hbm.at[idx_v.at[s]], bufs[b], gsems[b])

        def wait_gather(b):
            pltpu.make_async_copy(
                out_hbm.at[0, pl.ds(0, cols_per_w)], bufs[b], gsems[b]).wait()

        def wait_out(b):
            pltpu.make_async_copy(
                bufs[b], out_hbm.at[0, pl.ds(0, cols_per_w)], osems[b]).wait()

        for b in range(LA):
            issue_gather(b, b)

        def scale_buf(buf):
            def scale_row(r, c):
                for j in range(D_MODEL // L):
                    buf[r, pl.ds(j * L, L)] = buf[r, pl.ds(j * L, L)] * SCALE
                return c
            lax.fori_loop(0, cols_per_w, scale_row, 0)

        def outer(g2, carry):
            for b in range(NBUF):
                g = g2 * NBUF + b
                bl = (b + LA) % NBUF
                gl = g + LA

                @pl.when(gl < seq)
                def _issue():
                    @pl.when(gl >= NBUF)
                    def _wait_out():
                        wait_out(bl)
                    issue_gather(gl, bl)

                wait_gather(b)
                scale_buf(bufs[b])
                pltpu.async_copy(
                    bufs[b], out_hbm.at[g, pl.ds(col0, cols_per_w)], osems[b])
            return carry

        lax.fori_loop(0, seq // NBUF, outer, 0)

        for b in range(NBUF):
            wait_out(b)

    return k


@jax.jit
def kernel(x, table):
    n_rows, seq = x.shape
    xt = x.astype(jnp.int32).T
    out = _make_kernel(n_rows, seq)(xt, table)
    return out.transpose(1, 0, 2)


# speedup vs baseline: 1.0016x; 1.0016x over previous
import functools
import math

import jax
import jax.numpy as jnp
from jax import lax
from jax.experimental import pallas as pl
from jax.experimental.pallas import tpu as pltpu
from jax.experimental.pallas import tpu_sc as plsc

D_MODEL = 128
SCALE = math.sqrt(D_MODEL)

_info = plsc.get_sparse_core_info()
NC, NS, L = _info.num_cores, _info.num_subcores, _info.num_lanes
NW = NC * NS

NBUF = 5
LA = 2


def _make_kernel(n_rows, seq):
    assert n_rows % NW == 0
    cols_per_w = n_rows // NW
    assert seq % NBUF == 0
    mesh = plsc.VectorSubcoreMesh(core_axis_name="c", subcore_axis_name="s")

    @functools.partial(
        pl.kernel,
        mesh=mesh,
        out_type=jax.ShapeDtypeStruct((seq, n_rows, D_MODEL), jnp.float32),
        compiler_params=pltpu.CompilerParams(use_tc_tiling_on_sc=True),
        scratch_types=(
            [pltpu.VMEM((seq, cols_per_w), jnp.int32)]
            + [pltpu.VMEM((cols_per_w, D_MODEL), jnp.float32)] * NBUF
            + [pltpu.SemaphoreType.DMA] * (2 * NBUF)
        ),
    )
    def k(xt_hbm, table_hbm, out_hbm, idx_v, *rest):
        bufs = rest[:NBUF]
        gsems = rest[NBUF:2 * NBUF]
        osems = rest[2 * NBUF:3 * NBUF]
        wid = lax.axis_index("s") * NC + lax.axis_index("c")
        col0 = wid * cols_per_w
        pltpu.sync_copy(xt_hbm.at[:, pl.ds(col0, cols_per_w)], idx_v)

        def issue_gather(s, b):
            pltpu.async_copy(table_hbm.at[idx_v.at[s]], bufs[b], gsems[b])

        def wait_gather(b):
            pltpu.make_async_copy(
                out_hbm.at[0, pl.ds(0, cols_per_w)], bufs[b], gsems[b]).wait()

        def wait_out(b):
            pltpu.make_async_copy(
                bufs[b], out_hbm.at[0, pl.ds(0, cols_per_w)], osems[b]).wait()

        for b in range(LA):
            issue_gather(b, b)

        def scale_buf(buf):
            def scale_row(r, c):
                for j in range(D_MODEL // L):
                    buf[r, pl.ds(j * L, L)] = buf[r, pl.ds(j * L, L)] * SCALE
                return c
            lax.fori_loop(0, cols_per_w, scale_row, 0)

        def outer(g2, carry):
            for b in range(NBUF):
                g = g2 * NBUF + b
                bl = (b + LA) % NBUF
                gl = g + LA

                @pl.when(gl < seq)
                def _issue():
                    @pl.when(gl >= NBUF)
                    def _wait_out():
                        wait_out(bl)
                    issue_gather(gl, bl)

                wait_gather(b)
                scale_buf(bufs[b])
                pltpu.async_copy(
                    bufs[b], out_hbm.at[g, pl.ds(col0, cols_per_w)], osems[b])
            return carry

        lax.fori_loop(0, seq // NBUF, outer, 0)

        for b in range(NBUF):
            wait_out(b)

    return k


@jax.jit
def kernel(x, table):
    n_rows, seq = x.shape
    xt = x.astype(jnp.int32).T
    out = _make_kernel(n_rows, seq)(xt, table)
    return out.transpose(1, 0, 2)
